# Initial kernel scaffold; baseline (speedup 1.0000x reference)
import jax
import jax.numpy as jnp
from jax.experimental import pallas as pl


def kernel(x, edge_index_hetero, edge_index, Wih, Whh, bih, bhh, W1, a_src1, a_dst1, b1, W2, a_src2, a_dst2, b2, Wd, bd):
    raise NotImplementedError("write your pallas kernel here")


# speedup vs baseline: 24.8551x; 24.8551x over previous
import functools

import jax
import jax.numpy as jnp
from jax import lax
from jax.experimental import pallas as pl
from jax.experimental.pallas import tpu as pltpu
from jax.experimental.pallas import tpu_sc as plsc

N = 10000
T = 10
H = 64
NB = 1000
CH = 128
NW = 32
ZR = N // 16


def _enc_body(x_ref, wih_ref, whh_ref, bih_ref, bhh_ref, w1_ref, as1_ref,
              ad1_ref, w2_ref, as2_ref, ad2_ref,
              h1_ref, s1_ref, d1_ref, tx_ref, h2_ref, s2_ref, d2_ref):
    xb = x_ref[...]
    wih = wih_ref[...]
    whh = whh_ref[...]
    b = bih_ref[...] + bhh_ref[...]
    w1 = w1_ref[...]
    a_s1 = as1_ref[...]
    a_d1 = ad1_ref[...]
    w2 = w2_ref[...]
    a_s2 = as2_ref[...]
    a_d2 = ad2_ref[...]

    h = jnp.zeros((NB, H), jnp.float32)
    c = jnp.zeros((NB, H), jnp.float32)
    for t in range(T):
        xt = xb[:, t, :]
        g = (jnp.dot(xt, wih, preferred_element_type=jnp.float32)
             + jnp.dot(h, whh, preferred_element_type=jnp.float32) + b)
        ig = jax.nn.sigmoid(g[:, 0:H])
        fg = jax.nn.sigmoid(g[:, H:2 * H])
        gg = jnp.tanh(g[:, 2 * H:3 * H])
        og = jax.nn.sigmoid(g[:, 3 * H:4 * H])
        c = fg * c + ig * gg
        h = og * jnp.tanh(c)
        h1t = jnp.dot(h, w1, preferred_element_type=jnp.float32)
        h1_ref[t] = h1t
        s1_ref[t] = jnp.sum(h1t * a_s1, axis=-1, keepdims=True)
        d1_ref[t] = jnp.sum(h1t * a_d1, axis=-1, keepdims=True)

    tx_ref[...] = h
    h2t = jnp.dot(h, w2, preferred_element_type=jnp.float32)
    h2_ref[...] = h2t
    s2_ref[...] = jnp.sum(h2t * a_s2, axis=-1, keepdims=True)
    d2_ref[...] = jnp.sum(h2t * a_d2, axis=-1, keepdims=True)


def _full2(shape):
    return pl.BlockSpec(shape, lambda i: (0, 0))


_enc_call = pl.pallas_call(
    _enc_body,
    grid=(N // NB,),
    in_specs=[
        pl.BlockSpec((NB, T, H), lambda i: (i, 0, 0)),
        _full2((H, 4 * H)),
        _full2((H, 4 * H)),
        _full2((1, 4 * H)),
        _full2((1, 4 * H)),
        _full2((H, H)),
        _full2((1, H)),
        _full2((1, H)),
        _full2((H, H)),
        _full2((1, H)),
        _full2((1, H)),
    ],
    out_specs=[
        pl.BlockSpec((T, NB, H), lambda i: (0, i, 0)),
        pl.BlockSpec((T, NB, 1), lambda i: (0, i, 0)),
        pl.BlockSpec((T, NB, 1), lambda i: (0, i, 0)),
        pl.BlockSpec((NB, H), lambda i: (i, 0)),
        pl.BlockSpec((NB, H), lambda i: (i, 0)),
        pl.BlockSpec((NB, 1), lambda i: (i, 0)),
        pl.BlockSpec((NB, 1), lambda i: (i, 0)),
    ],
    out_shape=[
        jax.ShapeDtypeStruct((T, N, H), jnp.float32),
        jax.ShapeDtypeStruct((T, N, 1), jnp.float32),
        jax.ShapeDtypeStruct((T, N, 1), jnp.float32),
        jax.ShapeDtypeStruct((N, H), jnp.float32),
        jax.ShapeDtypeStruct((N, H), jnp.float32),
        jax.ShapeDtypeStruct((N, 1), jnp.float32),
        jax.ShapeDtypeStruct((N, 1), jnp.float32),
    ],
)


def _make_sc_body(n_edges, hetero):
    def _sc_body(hf, asf, adf, src_hbm, dst_hbm, acc_o, den_o,
                 sbuf, dbuf, idxs, idxd, rowb, mbuf, asb, adb, hrows, prod,
                 denr, wbuf, zb, zbd, acc_sh, den_sh, sem):
        cid = lax.axis_index("c")
        sid = lax.axis_index("s")
        wid = cid * 16 + sid
        lane0 = lax.iota(jnp.int32, 16) == 0

        def _zrow(j, carry):
            for cc in range(H // 16):
                zb[j, pl.ds(cc * 16, 16)] = jnp.zeros((16,), jnp.float32)
            zbd[j, pl.ds(0, 16)] = jnp.zeros((16,), jnp.float32)
            return carry

        lax.fori_loop(0, ZR, _zrow, 0)
        r0 = sid * ZR
        pltpu.sync_copy(zb, acc_sh.at[pl.ds(r0, ZR)])
        pltpu.sync_copy(zbd, den_sh.at[pl.ds(r0, ZR)])
        plsc.subcore_barrier()

        n_chunks = n_edges // CH
        nk = (n_chunks - wid + NW - 1) // NW

        def _chunk(k, carry):
            base = (wid + k * NW) * CH
            pltpu.sync_copy(src_hbm.at[pl.ds(base, CH)], sbuf)
            pltpu.sync_copy(dst_hbm.at[pl.ds(base, CH)], dbuf)
            for j in range(CH // 16):
                sl = pl.ds(j * 16, 16)
                sv = sbuf[sl]
                dv = dbuf[sl]
                if hetero:
                    sq = ((sv.astype(jnp.float32) + 0.5)
                          * (1.0 / T)).astype(jnp.int32)
                    sr = sv - sq * T
                    dq = ((dv.astype(jnp.float32) + 0.5)
                          * (1.0 / T)).astype(jnp.int32)
                    dr = dv - dq * T
                    idxs[sl] = sr * N + sq
                    idxd[sl] = dr * N + dq
                    rowb[sl] = dq
                    mbuf[sl] = jnp.where(dr == (T - 1), 1.0, 0.0)
                else:
                    idxs[sl] = sv
                    idxd[sl] = dv
                    rowb[sl] = dv
            pltpu.async_copy(asf.at[idxs], asb, sem).wait()
            pltpu.async_copy(adf.at[idxd], adb, sem).wait()
            pltpu.async_copy(hf.at[idxs], hrows, sem).wait()
            for j in range(CH // 16):
                sl = pl.ds(j * 16, 16)
                e = asb[sl] + adb[sl]
                e = jnp.where(e > 0.0, e, 0.2 * e)
                wt = jnp.exp(e)
                if hetero:
                    wt = jnp.where(mbuf[sl] > 0.0, wt, 0.0)
                wbuf[sl] = wt

            def _row(r, carry2):
                wrow = wbuf[pl.ds(r, 16)]
                wbc = jnp.take_along_axis(
                    wrow, jnp.zeros((16,), jnp.int32), axis=0)
                for cc in range(H // 16):
                    csl = pl.ds(cc * 16, 16)
                    prod[r, csl] = wbc * hrows[r, csl]
                denr[r, pl.ds(0, 16)] = jnp.where(lane0, wbc, 0.0)
                return carry2

            lax.fori_loop(0, CH, _row, 0)
            pltpu.sync_copy(prod, acc_sh.at[rowb], add=True)
            pltpu.sync_copy(denr, den_sh.at[rowb], add=True)
            return carry

        lax.fori_loop(0, nk, _chunk, 0)

        plsc.subcore_barrier()
        pltpu.sync_copy(acc_sh.at[pl.ds(r0, ZR)], acc_o.at[wid])
        pltpu.sync_copy(den_sh.at[pl.ds(r0, ZR)], den_o.at[wid])

    return _sc_body


def _make_gat_call(n_edges, hetero):
    return pl.kernel(
        _make_sc_body(n_edges, hetero),
        out_type=(
            jax.ShapeDtypeStruct((NW, ZR, H), jnp.float32),
            jax.ShapeDtypeStruct((NW, ZR, 16), jnp.float32),
        ),
        mesh=plsc.VectorSubcoreMesh(core_axis_name="c", subcore_axis_name="s"),
        compiler_params=pltpu.CompilerParams(use_tc_tiling_on_sc=False),
        scratch_types=[
            pltpu.VMEM((CH,), jnp.int32),
            pltpu.VMEM((CH,), jnp.int32),
            pltpu.VMEM((CH,), jnp.int32),
            pltpu.VMEM((CH,), jnp.int32),
            pltpu.VMEM((CH,), jnp.int32),
            pltpu.VMEM((CH,), jnp.float32),
            pltpu.VMEM((CH,), jnp.float32),
            pltpu.VMEM((CH,), jnp.float32),
            pltpu.VMEM((CH, H), jnp.float32),
            pltpu.VMEM((CH, H), jnp.float32),
            pltpu.VMEM((CH, 16), jnp.float32),
            pltpu.VMEM((CH + 16,), jnp.float32),
            pltpu.VMEM((ZR, H), jnp.float32),
            pltpu.VMEM((ZR, 16), jnp.float32),
            pltpu.VMEM_SHARED((N, H), jnp.float32),
            pltpu.VMEM_SHARED((N, 16), jnp.float32),
            pltpu.SemaphoreType.DMA,
        ],
    )


_gat1_call = _make_gat_call(1200000, True)
_gat2_call = _make_gat_call(160000, False)


def _dec_body(tx_ref, h19_ref, as19_ref, ad19_ref, h2_ref, as2_ref, ad2_ref,
              acc1_ref, den1_ref, acc2_ref, den2_ref, b1_ref, b2_ref,
              wd_ref, bd_ref, out_ref):
    tx = tx_ref[...]
    h19 = h19_ref[0]
    e1 = as19_ref[0] + ad19_ref[0]
    e1 = jnp.where(e1 > 0.0, e1, 0.2 * e1)
    w1s = jnp.exp(e1)
    num1 = acc1_ref[0] + acc1_ref[1] + w1s * h19
    den1 = den1_ref[0, :, 0:1] + den1_ref[1, :, 0:1] + w1s
    rx = num1 / (den1 + 1e-16) + b1_ref[...]

    h2v = h2_ref[...]
    e2 = as2_ref[...] + ad2_ref[...]
    e2 = jnp.where(e2 > 0.0, e2, 0.2 * e2)
    w2s = jnp.exp(e2)
    num2 = acc2_ref[0] + acc2_ref[1] + w2s * h2v
    den2 = den2_ref[0, :, 0:1] + den2_ref[1, :, 0:1] + w2s
    rxc = num2 / (den2 + 1e-16) + b2_ref[...]

    cat = jnp.concatenate([tx, rx, rxc], axis=-1)
    out_ref[...] = (jnp.dot(cat, wd_ref[...], preferred_element_type=jnp.float32)
                    + bd_ref[...])


_dec_call = pl.pallas_call(
    _dec_body,
    grid=(N // NB,),
    in_specs=[
        pl.BlockSpec((NB, H), lambda i: (i, 0)),
        pl.BlockSpec((1, NB, H), lambda i: (T - 1, i, 0)),
        pl.BlockSpec((1, NB, 1), lambda i: (T - 1, i, 0)),
        pl.BlockSpec((1, NB, 1), lambda i: (T - 1, i, 0)),
        pl.BlockSpec((NB, H), lambda i: (i, 0)),
        pl.BlockSpec((NB, 1), lambda i: (i, 0)),
        pl.BlockSpec((NB, 1), lambda i: (i, 0)),
        pl.BlockSpec((2, NB, H), lambda i: (0, i, 0)),
        pl.BlockSpec((2, NB, 16), lambda i: (0, i, 0)),
        pl.BlockSpec((2, NB, H), lambda i: (0, i, 0)),
        pl.BlockSpec((2, NB, 16), lambda i: (0, i, 0)),
        _full2((1, H)),
        _full2((1, H)),
        _full2((3 * H, 1)),
        _full2((1, 1)),
    ],
    out_specs=pl.BlockSpec((NB, 1), lambda i: (i, 0)),
    out_shape=jax.ShapeDtypeStruct((N, 1), jnp.float32),
)


def kernel(x, edge_index_hetero, edge_index, Wih, Whh, bih, bhh, W1, a_src1,
           a_dst1, b1, W2, a_src2, a_dst2, b2, Wd, bd):
    h1t, as1t, ad1t, t_x, h2, as2, ad2 = _enc_call(
        x, Wih.T, Whh.T, bih.reshape(1, -1), bhh.reshape(1, -1),
        W1, a_src1.reshape(1, -1), a_dst1.reshape(1, -1),
        W2, a_src2.reshape(1, -1), a_dst2.reshape(1, -1))

    acc1, den1 = _gat1_call(
        h1t.reshape(T * N, H), as1t.reshape(T * N), ad1t.reshape(T * N),
        edge_index_hetero[0], edge_index_hetero[1])
    acc2, den2 = _gat2_call(
        h2, as2.reshape(N), ad2.reshape(N),
        edge_index[0], edge_index[1])
    acc1 = acc1.reshape(2, N, H)
    den1 = den1.reshape(2, N, 16)
    acc2 = acc2.reshape(2, N, H)
    den2 = den2.reshape(2, N, 16)

    return _dec_call(t_x, h1t, as1t, ad1t, h2, as2, ad2,
                     acc1, den1, acc2, den2,
                     b1.reshape(1, -1), b2.reshape(1, -1), Wd,
                     bd.reshape(1, 1))
